# baseline (speedup 1.0000x reference)
import functools

import jax
import jax.numpy as jnp
from jax import lax
from jax.experimental import pallas as pl
from jax.experimental.pallas import tpu as pltpu
from jax.experimental.pallas import tpu_sc as plsc

N_NODES = 50000
N_EDGES = 800000
EMBED = 30
D = 32

NW = 32
GCHUNK = 128
CH = 512
CH_ROWS = CH // GCHUNK
NSLOT = 2
NCHUNK = 50
PER_W = CH * NCHUNK
PAD_E = PER_W * NW
LANES = 16

EPR = 4
ROWL = EPR * D
PROW = CH // EPR
BLK_E = 16000
RB = BLK_E // EPR
G = N_EDGES // BLK_E


def _sc_gather_mul(table, epad):
    mesh = plsc.VectorSubcoreMesh(core_axis_name="c", subcore_axis_name="s")

    @functools.partial(
        pl.kernel,
        mesh=mesh,
        compiler_params=pltpu.CompilerParams(use_tc_tiling_on_sc=False,
                                             needs_layout_passes=False),
        out_type=jax.ShapeDtypeStruct((PAD_E // EPR, ROWL), jnp.bfloat16),
        scratch_types=(
            [pltpu.VMEM((NSLOT, CH_ROWS, GCHUNK), jnp.int32)] * 2
            + [pltpu.VMEM((2 * GCHUNK,), jnp.int32)]
            + [pltpu.VMEM((CH, D), jnp.bfloat16)] * (2 * NSLOT)
            + [pltpu.VMEM((PROW, ROWL), jnp.bfloat16)] * NSLOT
            + [pltpu.SemaphoreType.DMA] * (2 * NSLOT)
        ),
    )
    def k(table_hbm, edges_hbm, out_hbm, sidx, didx, ebuf, *bufs):
        wid = lax.axis_index("s") * 2 + lax.axis_index("c")
        srows = bufs[0:NSLOT]
        drows = bufs[NSLOT:2 * NSLOT]
        pbuf = bufs[2 * NSLOT:3 * NSLOT]
        sg = bufs[3 * NSLOT:4 * NSLOT]
        sw = bufs[4 * NSLOT:5 * NSLOT]
        lane2 = jnp.arange(LANES, dtype=jnp.int32) * 2

        def issue(c, slot):
            ebase = (wid * NCHUNK + c) * CH
            for j in range(CH_ROWS):
                pltpu.sync_copy(
                    edges_hbm.at[pl.ds((ebase + j * GCHUNK) * 2, 2 * GCHUNK)],
                    ebuf)
                for g in range(GCHUNK // LANES):
                    rid = lane2 + (2 * g * LANES)
                    sidx[slot, j, pl.ds(g * LANES, LANES)] = (
                        plsc.load_gather(ebuf, [rid]))
                    didx[slot, j, pl.ds(g * LANES, LANES)] = (
                        plsc.load_gather(ebuf, [rid + 1]))
            for j in range(CH_ROWS):
                pltpu.async_copy(table_hbm.at[sidx.at[slot, j]],
                                 srows[slot].at[pl.ds(j * GCHUNK, GCHUNK)],
                                 sg[slot])
                pltpu.async_copy(table_hbm.at[didx.at[slot, j]],
                                 drows[slot].at[pl.ds(j * GCHUNK, GCHUNK)],
                                 sg[slot])

        def wait_gathers(slot):
            for j in range(CH_ROWS):
                pltpu.make_async_copy(
                    table_hbm.at[sidx.at[slot, j]],
                    srows[slot].at[pl.ds(j * GCHUNK, GCHUNK)], sg[slot]).wait()
                pltpu.make_async_copy(
                    table_hbm.at[didx.at[slot, j]],
                    drows[slot].at[pl.ds(j * GCHUNK, GCHUNK)], sg[slot]).wait()

        def drain_wb(slot):
            pltpu.make_async_copy(
                pbuf[slot], out_hbm.at[pl.ds(0, PROW)], sw[slot]).wait()

        def step(c, slot):
            nc = c + (NSLOT - 1)
            nslot = (slot + NSLOT - 1) % NSLOT

            @pl.when(nc < NCHUNK)
            def _():
                @pl.when(nc >= NSLOT)
                def _():
                    drain_wb(nslot)

                issue(nc, nslot)

            wait_gathers(slot)
            sr, dr, pb = srows[slot], drows[slot], pbuf[slot]

            def mul_body(m, c2):
                for u in range(EPR):
                    e_sl = (m * EPR + u, pl.ds(0, D))
                    pb[m, pl.ds(u * D, D)] = sr[e_sl] * dr[e_sl]
                return c2

            lax.fori_loop(0, PROW, mul_body, 0, unroll=False)
            rbase = (wid * NCHUNK + c) * PROW
            pltpu.async_copy(pb, out_hbm.at[pl.ds(rbase, PROW)], sw[slot])

        for p in range(NSLOT - 1):
            issue(p, p)

        def ring_body(i, carry):
            for p in range(NSLOT):
                step(NSLOT * i + p, p)
            return carry

        lax.fori_loop(0, NCHUNK // NSLOT, ring_body, 0, unroll=False)
        for p in range(NSLOT):
            drain_wb(p)

    return k(table, epad)


def _tc_mlp_loss(x128, labr, w1big, b1big, wbig, selm, carr):
    def body(x_ref, lab_ref, w1_ref, b1_ref, w_ref, sel_ref, c_ref, acc_ref):
        i = pl.program_id(0)
        xb = x_ref[...].astype(jnp.float32)
        h = jnp.dot(xb, w1_ref[...], preferred_element_type=jnp.float32)
        h = jnp.maximum(h + b1_ref[...], 0.0)
        s = h * w_ref[...]
        d = jnp.dot(s, sel_ref[...],
                    preferred_element_type=jnp.float32) + c_ref[...]
        t = 1.0 / (1.0 + jnp.exp(-d))
        y = jnp.log(jnp.exp(t) + jnp.exp(1.0 - t))
        lf = lab_ref[0]
        sl = t + lf * (1.0 - 2.0 * t)
        part = jnp.sum(y - sl).reshape(1, 1)

        @pl.when(i == 0)
        def _():
            acc_ref[...] = jnp.zeros((1, 1), jnp.float32)

        acc_ref[...] += part

    return pl.pallas_call(
        body,
        grid=(G,),
        in_specs=[
            pl.BlockSpec((RB, ROWL), lambda i: (i, 0)),
            pl.BlockSpec((1, RB, EPR), lambda i: (i, 0, 0)),
            pl.BlockSpec((ROWL, ROWL), lambda i: (0, 0)),
            pl.BlockSpec((1, ROWL), lambda i: (0, 0)),
            pl.BlockSpec((1, ROWL), lambda i: (0, 0)),
            pl.BlockSpec((ROWL, EPR), lambda i: (0, 0)),
            pl.BlockSpec((1, 1), lambda i: (0, 0)),
        ],
        out_specs=pl.BlockSpec((1, 1), lambda i: (0, 0)),
        out_shape=jax.ShapeDtypeStruct((1, 1), jnp.float32),
    )(x128, labr, w1big, b1big, wbig, selm, carr)


def kernel(edges, labels, word_embeddings, W1, b1, W2, b2):
    epad = jnp.pad(edges.astype(jnp.int32),
                   ((0, PAD_E - N_EDGES), (0, 0))).reshape(-1)
    table = jnp.pad(word_embeddings.astype(jnp.float32),
                    ((0, 0), (0, D - EMBED))).astype(jnp.bfloat16)

    w1p = jnp.pad(W1.astype(jnp.float32), ((0, D - EMBED), (0, D - EMBED)))
    seg = (jnp.arange(ROWL, dtype=jnp.int32) // D)
    blockmask = (seg[:, None] == seg[None, :]).astype(jnp.float32)
    w1big = jnp.tile(w1p, (EPR, EPR)) * blockmask
    b1big = jnp.tile(jnp.pad(b1.astype(jnp.float32), (0, D - EMBED)),
                     EPR).reshape(1, ROWL)
    wbig = jnp.tile(jnp.pad((W2[:, 0] - W2[:, 1]).astype(jnp.float32),
                            (0, D - EMBED)), EPR).reshape(1, ROWL)
    selm = (seg[:, None] == jnp.arange(EPR, dtype=jnp.int32)[None, :]
            ).astype(jnp.float32)
    carr = (b2[0] - b2[1]).astype(jnp.float32).reshape(1, 1)
    labr = labels.astype(jnp.float32).reshape(G, RB, EPR)

    x128 = _sc_gather_mul(table, epad)

    acc = _tc_mlp_loss(x128, labr, w1big, b1big, wbig, selm, carr)
    return (acc[0, 0] / jnp.float32(N_EDGES)).astype(jnp.float32)


# speedup vs baseline: 2.8061x; 2.8061x over previous
import functools

import jax
import jax.numpy as jnp
from jax import lax
from jax.experimental import pallas as pl
from jax.experimental.pallas import tpu as pltpu
from jax.experimental.pallas import tpu_sc as plsc

N_NODES = 50000
N_EDGES = 800000
EMBED = 30
D = 32

NW = 32
GCHUNK = 128
CH = 512
CH_ROWS = CH // GCHUNK
NSLOT = 2
NCHUNK = 50
PER_W = CH * NCHUNK
PAD_E = PER_W * NW
LANES = 16

EPR = 4
ROWL = EPR * D
PROW = CH // EPR
BLK_E = 16000
RB = BLK_E // EPR
G = N_EDGES // BLK_E


def _sc_gather_mul(table, src2d, dst2d):
    mesh = plsc.VectorSubcoreMesh(core_axis_name="c", subcore_axis_name="s")

    @functools.partial(
        pl.kernel,
        mesh=mesh,
        compiler_params=pltpu.CompilerParams(use_tc_tiling_on_sc=False),
        out_type=jax.ShapeDtypeStruct((PAD_E // EPR, ROWL), jnp.bfloat16),
        scratch_types=(
            [pltpu.VMEM((NSLOT, CH_ROWS, GCHUNK), jnp.int32)] * 2
            + [pltpu.VMEM((CH, D), jnp.bfloat16)] * (2 * NSLOT)
            + [pltpu.VMEM((PROW, ROWL), jnp.bfloat16)] * NSLOT
            + [pltpu.SemaphoreType.DMA] * (2 * NSLOT)
        ),
    )
    def k(table_hbm, src_hbm, dst_hbm, out_hbm, sidx, didx, *bufs):
        wid = lax.axis_index("s") * 2 + lax.axis_index("c")
        srows = bufs[0:NSLOT]
        drows = bufs[NSLOT:2 * NSLOT]
        pbuf = bufs[2 * NSLOT:3 * NSLOT]
        sg = bufs[3 * NSLOT:4 * NSLOT]
        sw = bufs[4 * NSLOT:5 * NSLOT]
        def issue(c, slot):
            crow = (wid * NCHUNK + c) * CH_ROWS
            pltpu.sync_copy(src_hbm.at[pl.ds(crow, CH_ROWS)], sidx.at[slot])
            pltpu.sync_copy(dst_hbm.at[pl.ds(crow, CH_ROWS)], didx.at[slot])
            for j in range(CH_ROWS):
                pltpu.async_copy(table_hbm.at[sidx.at[slot, j]],
                                 srows[slot].at[pl.ds(j * GCHUNK, GCHUNK)],
                                 sg[slot])
                pltpu.async_copy(table_hbm.at[didx.at[slot, j]],
                                 drows[slot].at[pl.ds(j * GCHUNK, GCHUNK)],
                                 sg[slot])

        def wait_gathers(slot):
            for j in range(CH_ROWS):
                pltpu.make_async_copy(
                    table_hbm.at[sidx.at[slot, j]],
                    srows[slot].at[pl.ds(j * GCHUNK, GCHUNK)], sg[slot]).wait()
                pltpu.make_async_copy(
                    table_hbm.at[didx.at[slot, j]],
                    drows[slot].at[pl.ds(j * GCHUNK, GCHUNK)], sg[slot]).wait()

        def drain_wb(slot):
            pltpu.make_async_copy(
                pbuf[slot], out_hbm.at[pl.ds(0, PROW)], sw[slot]).wait()

        def step(c, slot):
            nc = c + (NSLOT - 1)
            nslot = (slot + NSLOT - 1) % NSLOT

            @pl.when(nc < NCHUNK)
            def _():
                @pl.when(nc >= NSLOT)
                def _():
                    drain_wb(nslot)

                issue(nc, nslot)

            wait_gathers(slot)
            sr, dr, pb = srows[slot], drows[slot], pbuf[slot]

            def mul_body(m, c2):
                for u in range(EPR):
                    e_sl = (m * EPR + u, pl.ds(0, D))
                    pb[m, pl.ds(u * D, D)] = sr[e_sl] * dr[e_sl]
                return c2

            lax.fori_loop(0, PROW, mul_body, 0, unroll=False)
            rbase = (wid * NCHUNK + c) * PROW
            pltpu.async_copy(pb, out_hbm.at[pl.ds(rbase, PROW)], sw[slot])

        for p in range(NSLOT - 1):
            issue(p, p)

        def ring_body(i, carry):
            for p in range(NSLOT):
                step(NSLOT * i + p, p)
            return carry

        lax.fori_loop(0, NCHUNK // NSLOT, ring_body, 0, unroll=False)
        for p in range(NSLOT):
            drain_wb(p)

    return k(table, src2d, dst2d)


def _tc_mlp_loss(x128, labr, w1big, b1big, wbig, selm, carr):
    def body(x_ref, lab_ref, w1_ref, b1_ref, w_ref, sel_ref, c_ref, acc_ref):
        i = pl.program_id(0)
        xb = x_ref[...].astype(jnp.float32)
        h = jnp.dot(xb, w1_ref[...], preferred_element_type=jnp.float32)
        h = jnp.maximum(h + b1_ref[...], 0.0)
        s = h * w_ref[...]
        d = jnp.dot(s, sel_ref[...],
                    preferred_element_type=jnp.float32) + c_ref[...]
        t = 1.0 / (1.0 + jnp.exp(-d))
        y = jnp.log(jnp.exp(t) + jnp.exp(1.0 - t))
        lf = lab_ref[0]
        sl = t + lf * (1.0 - 2.0 * t)
        part = jnp.sum(y - sl).reshape(1, 1)

        @pl.when(i == 0)
        def _():
            acc_ref[...] = jnp.zeros((1, 1), jnp.float32)

        acc_ref[...] += part

    return pl.pallas_call(
        body,
        grid=(G,),
        in_specs=[
            pl.BlockSpec((RB, ROWL), lambda i: (i, 0)),
            pl.BlockSpec((1, RB, EPR), lambda i: (i, 0, 0)),
            pl.BlockSpec((ROWL, ROWL), lambda i: (0, 0)),
            pl.BlockSpec((1, ROWL), lambda i: (0, 0)),
            pl.BlockSpec((1, ROWL), lambda i: (0, 0)),
            pl.BlockSpec((ROWL, EPR), lambda i: (0, 0)),
            pl.BlockSpec((1, 1), lambda i: (0, 0)),
        ],
        out_specs=pl.BlockSpec((1, 1), lambda i: (0, 0)),
        out_shape=jax.ShapeDtypeStruct((1, 1), jnp.float32),
    )(x128, labr, w1big, b1big, wbig, selm, carr)


def kernel(edges, labels, word_embeddings, W1, b1, W2, b2):
    src = edges[:, 0].astype(jnp.int32)
    dst = edges[:, 1].astype(jnp.int32)
    pad = PAD_E - N_EDGES
    src2d = jnp.pad(src, (0, pad)).reshape(PAD_E // GCHUNK, GCHUNK)
    dst2d = jnp.pad(dst, (0, pad)).reshape(PAD_E // GCHUNK, GCHUNK)
    table = jnp.pad(word_embeddings.astype(jnp.float32),
                    ((0, 0), (0, D - EMBED))).astype(jnp.bfloat16)

    w1p = jnp.pad(W1.astype(jnp.float32), ((0, D - EMBED), (0, D - EMBED)))
    seg = (jnp.arange(ROWL, dtype=jnp.int32) // D)
    blockmask = (seg[:, None] == seg[None, :]).astype(jnp.float32)
    w1big = jnp.tile(w1p, (EPR, EPR)) * blockmask
    b1big = jnp.tile(jnp.pad(b1.astype(jnp.float32), (0, D - EMBED)),
                     EPR).reshape(1, ROWL)
    wbig = jnp.tile(jnp.pad((W2[:, 0] - W2[:, 1]).astype(jnp.float32),
                            (0, D - EMBED)), EPR).reshape(1, ROWL)
    selm = (seg[:, None] == jnp.arange(EPR, dtype=jnp.int32)[None, :]
            ).astype(jnp.float32)
    carr = (b2[0] - b2[1]).astype(jnp.float32).reshape(1, 1)
    labr = labels.astype(jnp.float32).reshape(G, RB, EPR)

    x128 = _sc_gather_mul(table, src2d, dst2d)

    acc = _tc_mlp_loss(x128, labr, w1big, b1big, wbig, selm, carr)
    return (acc[0, 0] / jnp.float32(N_EDGES)).astype(jnp.float32)
